# baseline (speedup 1.0000x reference)
import jax
import jax.numpy as jnp
from jax.experimental import pallas as pl
from jax.experimental.pallas import tpu as pltpu

_MASK_RATIO = 0.75


def _mask_kernel(noise_ref, mask_ref, *, k):
    bits = jax.lax.bitcast_convert_type(noise_ref[...], jnp.int32)
    d, l = bits.shape

    def body(_, carry):
        lo, hi = carry
        mid = lo + (hi - lo) // 2
        cnt = jnp.sum((bits <= mid).astype(jnp.int32), axis=1, keepdims=True)
        ge = cnt >= k
        return jnp.where(ge, lo, mid + 1), jnp.where(ge, mid, hi)

    lo = jnp.zeros((d, 1), jnp.int32)
    hi = jnp.full((d, 1), 1 << 30, jnp.int32)
    lo, hi = jax.lax.fori_loop(0, 30, body, (lo, hi))
    thresh = lo

    lt = bits < thresh
    eq = bits == thresh
    cnt_lt = jnp.sum(lt.astype(jnp.int32), axis=1, keepdims=True)
    ties_to_keep = (k - cnt_lt).astype(jnp.float32)

    row = jax.lax.broadcasted_iota(jnp.int32, (l, l), 0)
    col = jax.lax.broadcasted_iota(jnp.int32, (l, l), 1)
    tri = (row < col).astype(jnp.float32)
    prefix_eq = jax.lax.dot(eq.astype(jnp.float32), tri,
                            preferred_element_type=jnp.float32)
    keep = lt | (eq & (prefix_eq < ties_to_keep))
    mask_ref[...] = keep.astype(jnp.float32)


def _mul_kernel(x_hbm, mask_ref, out_hbm, inb, outb, isem, osem,
                *, bd, nbuf, nsteps):
    def in_copy(i, slot):
        return pltpu.make_async_copy(
            x_hbm.at[pl.ds(i * bd, bd)], inb.at[slot], isem.at[slot])

    def out_copy(i, slot):
        return pltpu.make_async_copy(
            outb.at[slot], out_hbm.at[pl.ds(i * bd, bd)], osem.at[slot])

    for j in range(min(nbuf, nsteps)):
        in_copy(j, j).start()

    def body(i, _):
        slot = jax.lax.rem(i, nbuf)
        in_copy(i, slot).wait()

        @pl.when(i >= nbuf)
        def _wait_out():
            out_copy(i - nbuf, slot).wait()

        m = mask_ref[i]
        outb[slot] = inb[slot] * m[:, None, :]
        out_copy(i, slot).start()

        @pl.when(i + nbuf < nsteps)
        def _next_in():
            in_copy(i + nbuf, slot).start()

        return 0

    jax.lax.fori_loop(0, nsteps, body, 0)

    for j in range(max(0, nsteps - nbuf), nsteps):
        out_copy(j, j % nbuf).wait()


def kernel(x, noise):
    d, c, h, w = x.shape
    l = h * w
    k = int(l * (1 - _MASK_RATIO))
    x3 = x.reshape(d, c, l)

    mask = pl.pallas_call(
        lambda nr, mr: _mask_kernel(nr, mr, k=k),
        out_shape=jax.ShapeDtypeStruct((d, l), jnp.float32),
    )(noise)

    bd = 4
    nbuf = 6
    nsteps = d // bd
    out3 = pl.pallas_call(
        lambda *refs: _mul_kernel(*refs, bd=bd, nbuf=nbuf, nsteps=nsteps),
        in_specs=[
            pl.BlockSpec(memory_space=pl.ANY),
            pl.BlockSpec(memory_space=pltpu.MemorySpace.VMEM),
        ],
        out_specs=pl.BlockSpec(memory_space=pl.ANY),
        out_shape=jax.ShapeDtypeStruct((d, c, l), x.dtype),
        scratch_shapes=[
            pltpu.VMEM((nbuf, bd, c, l), jnp.float32),
            pltpu.VMEM((nbuf, bd, c, l), jnp.float32),
            pltpu.SemaphoreType.DMA((nbuf,)),
            pltpu.SemaphoreType.DMA((nbuf,)),
        ],
    )(x3, mask.reshape(nsteps, bd, l))

    return out3.reshape(d, c, h, w)
